# baseline (speedup 1.0000x reference)
import functools

import jax
import jax.numpy as jnp
import numpy as np
from jax import lax
from jax.experimental import pallas as pl
from jax.experimental.pallas import tpu as pltpu
from jax.experimental.pallas import tpu_sc as plsc

B = 4096
C = 50
CPAD = 64
D = 256
TEMP = 0.01
NC = 2
NS = 16
L = 16
NW = NC * NS
RPW = B // NW
IPW = B // NS


def _rotl(x, r):
    return ((x << np.uint32(r)) | (x >> np.uint32(32 - r))).astype(np.uint32)


def _tf2x32(k0, k1, x0, x1):
    x0 = x0.astype(np.uint32).copy()
    x1 = x1.astype(np.uint32).copy()
    ks = [k0, k1, np.bitwise_xor(np.bitwise_xor(k0, k1),
                                 np.uint32(0x1BD11BDA)).astype(np.uint32)]
    rotations = [(13, 15, 26, 6), (17, 29, 16, 24)]
    x0 = (x0 + ks[0]).astype(np.uint32)
    x1 = (x1 + ks[1]).astype(np.uint32)
    for i in range(5):
        for r in rotations[i % 2]:
            x0 = (x0 + x1).astype(np.uint32)
            x1 = _rotl(x1, r) ^ x0
        x0 = (x0 + ks[(i + 1) % 3]).astype(np.uint32)
        x1 = (x1 + ks[(i + 2) % 3] + np.uint32(i + 1)).astype(np.uint32)
    return x0, x1


def _pair_words():
    z = np.zeros(B, np.uint32)
    b1, b2 = _tf2x32(np.uint32(0), np.uint32(1), z, np.arange(B, dtype=np.uint32))
    c1a, c2a = _tf2x32(b1, b2, z, z)
    c1b, c2b = _tf2x32(b1, b2, z, np.ones(B, np.uint32))
    h1, h2 = _tf2x32(c1a, c2a, z, z)
    l1, l2 = _tf2x32(c1b, c2b, z, z)
    return (h1 ^ h2).view(np.int32), (l1 ^ l2).view(np.int32)


_HI_W, _LO_W = _pair_words()


def _sc_body(labels_hbm, hi_hbm, lo_hbm, pre_hbm, out_hbm,
             labels_v, lrow_v, rank_v, grow_v, m_v, hist_v, offs_v, pret_v,
             allh_v, hiw_v, low_v, jidx_v, qa_v, qb_v, ida_v, idb_v, rows_v,
             prow_v, tr_v, ss_v, hs_s, rs_s, ms_s, sem, sem2, sem3):
    cid = lax.axis_index("c")
    sid = lax.axis_index("s")
    wid = cid * NS + sid
    rbase = wid * RPW
    ibase = sid * IPW

    pltpu.sync_copy(labels_hbm.at[pl.ds(ibase, IPW)], labels_v)
    pltpu.sync_copy(labels_hbm.at[pl.ds(rbase, RPW)], lrow_v)
    pltpu.sync_copy(hi_hbm.at[pl.ds(rbase, RPW)], hiw_v)
    pltpu.sync_copy(lo_hbm.at[pl.ds(rbase, RPW)], low_v)
    own_rows = pltpu.async_copy(pre_hbm.at[pl.ds(rbase, RPW)], prow_v, sem2)

    lane = lax.iota(jnp.int32, L)
    zeros = jnp.zeros((L,), jnp.int32)
    for c in range(CPAD // L):
        hist_v[pl.ds(c * L, L)] = zeros

    for k in range(IPW // L):
        lbl = labels_v[pl.ds(k * L, L)]
        occ, last = plsc.scan_count(lbl)
        h = plsc.load_gather(hist_v, [lbl])
        rank_v[pl.ds(k * L, L)] = h + occ - 1
        plsc.store_scatter(hist_v, [lbl], h + occ, mask=last)

    pltpu.sync_copy(hist_v, hs_s.at[sid])
    plsc.subcore_barrier()
    pltpu.sync_copy(hs_s, allh_v)
    carry = jnp.int32(0)
    for c in range(CPAD // L):
        tot = zeros
        pre = zeros
        for t in range(NS):
            row = allh_v[t, pl.ds(c * L, L)]
            tot = tot + row
            pre = pre + jnp.where(t < sid, row, zeros)
        hist_v[pl.ds(c * L, L)] = tot
        pret_v[pl.ds(c * L, L)] = pre
        cum = plsc.cumsum(tot)
        offs_v[pl.ds(c * L, L)] = cum - tot + carry
        carry = carry + jnp.sum(tot)

    for k in range(IPW // L):
        lbl = labels_v[pl.ds(k * L, L)]
        g = rank_v[pl.ds(k * L, L)] + plsc.load_gather(pret_v, [lbl])
        rank_v[pl.ds(k * L, L)] = g
        q = plsc.load_gather(offs_v, [lbl]) + g
        half, off = divmod(k * L, RPW)
        qref, idref = (qa_v, ida_v) if half == 0 else (qb_v, idb_v)
        qref[pl.ds(off, L)] = q
        idref[pl.ds(off, L)] = ibase + k * L + lane
    pltpu.sync_copy(rank_v, rs_s.at[pl.ds(ibase, IPW)])
    pltpu.sync_copy(ida_v, ms_s.at[qa_v])
    pltpu.sync_copy(idb_v, ms_s.at[qb_v])
    plsc.subcore_barrier()

    pltpu.sync_copy(ms_s, m_v)
    pltpu.sync_copy(rs_s.at[pl.ds(rbase, RPW)], grow_v)

    for k in range(RPW // L):
        lbl = lrow_v[pl.ds(k * L, L)]
        rnk = grow_v[pl.ds(k * L, L)]
        hi = hiw_v[pl.ds(k * L, L)]
        lo = low_v[pl.ds(k * L, L)]
        cnt = plsc.load_gather(hist_v, [lbl]) - 1
        span = jnp.maximum(cnt, 1)
        m1 = lax.rem(jnp.full((L,), 1 << 16, jnp.int32), span)
        mult = lax.rem(m1 * m1, span)

        def u32mod(w, span=span, m1=m1):
            wh = lax.shift_right_logical(w, 16)
            wl = jnp.bitwise_and(w, 0xFFFF)
            return lax.rem(lax.rem(wh, span) * m1 + lax.rem(wl, span), span)

        r = lax.rem(u32mod(hi) * mult + u32mod(lo), span)
        s = r + (r >= rnk).astype(jnp.int32)
        off = plsc.load_gather(offs_v, [lbl])
        pos = jnp.minimum(off + s, B - 1)
        j = plsc.load_gather(m_v, [pos])
        gi = rbase + k * L + lane
        j = jnp.where(cnt == 0, gi, j)
        jidx_v[pl.ds(k * L, L)] = j

    half = RPW // 2
    ga = pltpu.async_copy(pre_hbm.at[jidx_v.at[pl.ds(0, half)]],
                          rows_v.at[pl.ds(0, half), :], sem)
    gb = pltpu.async_copy(pre_hbm.at[jidx_v.at[pl.ds(half, half)]],
                          rows_v.at[pl.ds(half, half), :], sem3)

    def ssgroup(g, carry):
        for r in range(L):
            acc = jnp.zeros((L,), jnp.float32)
            for c in range(D // L):
                a = prow_v[g * L + r, pl.ds(c * L, L)]
                b = rows_v[g * L + r, pl.ds(c * L, L)]
                dlt = a - b
                acc = acc + dlt * dlt
            plsc.store_scatter(tr_v, [lane, jnp.full((L,), r, jnp.int32)], acc)
        tot = jnp.zeros((L,), jnp.float32)
        for r in range(L):
            tot = tot + tr_v[r, :]
        ss_v[pl.ds(g * L, L)] = tot
        return carry

    ngrp = RPW // L
    own_rows.wait()
    ga.wait()
    lax.fori_loop(0, ngrp // 2, ssgroup, 0)
    gb.wait()
    lax.fori_loop(ngrp // 2, ngrp, ssgroup, 0)
    pltpu.sync_copy(ss_v, out_hbm.at[wid])


def _sc_pair_gather(*args):
    return functools.partial(
        pl.kernel,
        out_type=jax.ShapeDtypeStruct((NW, RPW), jnp.float32),
        mesh=plsc.VectorSubcoreMesh(
            core_axis_name="c", subcore_axis_name="s",
            num_cores=NC, num_subcores=NS),
        compiler_params=pltpu.CompilerParams(needs_layout_passes=False),
        scratch_types=[
            pltpu.VMEM((IPW,), jnp.int32),
            pltpu.VMEM((RPW,), jnp.int32),
            pltpu.VMEM((IPW,), jnp.int32),
            pltpu.VMEM((RPW,), jnp.int32),
            pltpu.VMEM((B,), jnp.int32),
            pltpu.VMEM((CPAD,), jnp.int32),
            pltpu.VMEM((CPAD,), jnp.int32),
            pltpu.VMEM((CPAD,), jnp.int32),
            pltpu.VMEM((NS, CPAD), jnp.int32),
            pltpu.VMEM((RPW,), jnp.int32),
            pltpu.VMEM((RPW,), jnp.int32),
            pltpu.VMEM((RPW,), jnp.int32),
            pltpu.VMEM((RPW,), jnp.int32),
            pltpu.VMEM((RPW,), jnp.int32),
            pltpu.VMEM((RPW,), jnp.int32),
            pltpu.VMEM((RPW,), jnp.int32),
            pltpu.VMEM((RPW, D), jnp.float32),
            pltpu.VMEM((RPW, D), jnp.float32),
            pltpu.VMEM((L, L), jnp.float32),
            pltpu.VMEM((RPW,), jnp.float32),
            pltpu.VMEM_SHARED((NS, CPAD), jnp.int32),
            pltpu.VMEM_SHARED((B,), jnp.int32),
            pltpu.VMEM_SHARED((B,), jnp.int32),
            pltpu.SemaphoreType.DMA,
            pltpu.SemaphoreType.DMA,
            pltpu.SemaphoreType.DMA,
        ],
    )(_sc_body)(*args)


def _tc_body(post_t_ref, lab_ref, ss_ref, out_ref):
    ss = ss_ref[...]
    dist = jnp.sum(jnp.sqrt(jnp.where(ss == 0.0, 1e-5, ss)))
    x = post_t_ref[...]
    m = jnp.max(x, axis=0, keepdims=True)
    s = jnp.sum(jnp.exp(x - m), axis=0, keepdims=True)
    lse_sum = jnp.sum(jnp.log(s) + m)
    cls = lax.broadcasted_iota(jnp.int32, (C, 1), 0)
    onehot = lab_ref[...] == cls
    xl_sum = jnp.sum(jnp.where(onehot, x, 0.0))
    out_ref[...] = jnp.reshape((lse_sum - xl_sum) / B + TEMP * dist, (1, 1))


def kernel(pre_projection_activations, post_projection_activations, labels):
    pre = pre_projection_activations
    post = post_projection_activations
    lab32 = labels.astype(jnp.int32)
    hi = jnp.asarray(_HI_W)
    lo = jnp.asarray(_LO_W)
    ss = _sc_pair_gather(lab32, hi, lo, pre)
    out = pl.pallas_call(
        _tc_body,
        out_shape=jax.ShapeDtypeStruct((1, 1), jnp.float32),
    )(post.T, lab32.reshape(1, B), ss)
    return out[0, 0]


# speedup vs baseline: 1.0269x; 1.0269x over previous
import functools

import jax
import jax.numpy as jnp
import numpy as np
from jax import lax
from jax.experimental import pallas as pl
from jax.experimental.pallas import tpu as pltpu
from jax.experimental.pallas import tpu_sc as plsc

B = 4096
C = 50
CPAD = 64
D = 256
TEMP = 0.01
NC = 2
NS = 16
L = 16
NW = NC * NS
RPW = B // NW
IPW = B // NS


def _rotl(x, r):
    return ((x << np.uint32(r)) | (x >> np.uint32(32 - r))).astype(np.uint32)


def _tf2x32(k0, k1, x0, x1):
    x0 = x0.astype(np.uint32).copy()
    x1 = x1.astype(np.uint32).copy()
    ks = [k0, k1, np.bitwise_xor(np.bitwise_xor(k0, k1),
                                 np.uint32(0x1BD11BDA)).astype(np.uint32)]
    rotations = [(13, 15, 26, 6), (17, 29, 16, 24)]
    x0 = (x0 + ks[0]).astype(np.uint32)
    x1 = (x1 + ks[1]).astype(np.uint32)
    for i in range(5):
        for r in rotations[i % 2]:
            x0 = (x0 + x1).astype(np.uint32)
            x1 = _rotl(x1, r) ^ x0
        x0 = (x0 + ks[(i + 1) % 3]).astype(np.uint32)
        x1 = (x1 + ks[(i + 2) % 3] + np.uint32(i + 1)).astype(np.uint32)
    return x0, x1


def _pair_words():
    z = np.zeros(B, np.uint32)
    b1, b2 = _tf2x32(np.uint32(0), np.uint32(1), z, np.arange(B, dtype=np.uint32))
    c1a, c2a = _tf2x32(b1, b2, z, z)
    c1b, c2b = _tf2x32(b1, b2, z, np.ones(B, np.uint32))
    h1, h2 = _tf2x32(c1a, c2a, z, z)
    l1, l2 = _tf2x32(c1b, c2b, z, z)
    return (h1 ^ h2).view(np.int32), (l1 ^ l2).view(np.int32)


_HI_W, _LO_W = _pair_words()
_HL_W = np.concatenate(
    [_HI_W.reshape(NW, B // NW), _LO_W.reshape(NW, B // NW)], axis=1)


def _sc_body(labels_hbm, hl_hbm, pre_hbm, out_hbm,
             labels_v, rank_v, m_v, hist_v, offs_v, pret_v,
             allh_v, hlw_v, jidx_v, qa_v, qb_v, ida_v, idb_v, rows_v,
             prow_v, tr_v, ss_v, hs_s, ms_s, sem, sem2, sem3):
    cid = lax.axis_index("c")
    sid = lax.axis_index("s")
    ibase = sid * IPW
    obase = cid * RPW
    rbase = ibase + obase
    owid = sid * NC + cid

    pltpu.sync_copy(labels_hbm.at[pl.ds(ibase, IPW)], labels_v)
    pltpu.sync_copy(hl_hbm.at[owid], hlw_v)
    own_rows = pltpu.async_copy(pre_hbm.at[pl.ds(rbase, RPW)], prow_v, sem2)

    lane = lax.iota(jnp.int32, L)
    zeros = jnp.zeros((L,), jnp.int32)
    for c in range(CPAD // L):
        hist_v[pl.ds(c * L, L)] = zeros

    for k in range(IPW // L):
        lbl = labels_v[pl.ds(k * L, L)]
        occ, last = plsc.scan_count(lbl)
        h = plsc.load_gather(hist_v, [lbl])
        rank_v[pl.ds(k * L, L)] = h + occ - 1
        plsc.store_scatter(hist_v, [lbl], h + occ, mask=last)

    pltpu.sync_copy(hist_v, hs_s.at[sid])
    plsc.subcore_barrier()
    pltpu.sync_copy(hs_s, allh_v)
    carry = jnp.int32(0)
    for c in range(CPAD // L):
        tot = zeros
        pre = zeros
        for t in range(NS):
            row = allh_v[t, pl.ds(c * L, L)]
            tot = tot + row
            pre = pre + jnp.where(t < sid, row, zeros)
        hist_v[pl.ds(c * L, L)] = tot
        pret_v[pl.ds(c * L, L)] = pre
        cum = plsc.cumsum(tot)
        offs_v[pl.ds(c * L, L)] = cum - tot + carry
        carry = carry + jnp.sum(tot)

    for k in range(IPW // L):
        lbl = labels_v[pl.ds(k * L, L)]
        g = rank_v[pl.ds(k * L, L)] + plsc.load_gather(pret_v, [lbl])
        rank_v[pl.ds(k * L, L)] = g
        q = plsc.load_gather(offs_v, [lbl]) + g
        half, off = divmod(k * L, RPW)
        qref, idref = (qa_v, ida_v) if half == 0 else (qb_v, idb_v)
        qref[pl.ds(off, L)] = q
        idref[pl.ds(off, L)] = ibase + k * L + lane
    pltpu.sync_copy(ida_v, ms_s.at[qa_v])
    pltpu.sync_copy(idb_v, ms_s.at[qb_v])
    plsc.subcore_barrier()

    pltpu.sync_copy(ms_s, m_v)

    for k in range(RPW // L):
        lbl = labels_v[pl.ds(obase + k * L, L)]
        rnk = rank_v[pl.ds(obase + k * L, L)]
        hi = hlw_v[pl.ds(k * L, L)]
        lo = hlw_v[pl.ds(RPW + k * L, L)]
        cnt = plsc.load_gather(hist_v, [lbl]) - 1
        span = jnp.maximum(cnt, 1)
        m1 = lax.rem(jnp.full((L,), 1 << 16, jnp.int32), span)
        mult = lax.rem(m1 * m1, span)

        def u32mod(w, span=span, m1=m1):
            wh = lax.shift_right_logical(w, 16)
            wl = jnp.bitwise_and(w, 0xFFFF)
            return lax.rem(lax.rem(wh, span) * m1 + lax.rem(wl, span), span)

        r = lax.rem(u32mod(hi) * mult + u32mod(lo), span)
        s = r + (r >= rnk).astype(jnp.int32)
        off = plsc.load_gather(offs_v, [lbl])
        pos = jnp.minimum(off + s, B - 1)
        j = plsc.load_gather(m_v, [pos])
        gi = rbase + k * L + lane
        j = jnp.where(cnt == 0, gi, j)
        jidx_v[pl.ds(k * L, L)] = j

    half = RPW // 2
    ga = pltpu.async_copy(pre_hbm.at[jidx_v.at[pl.ds(0, half)]],
                          rows_v.at[pl.ds(0, half), :], sem)
    gb = pltpu.async_copy(pre_hbm.at[jidx_v.at[pl.ds(half, half)]],
                          rows_v.at[pl.ds(half, half), :], sem3)

    def ssgroup(g, carry):
        for r in range(L):
            acc = jnp.zeros((L,), jnp.float32)
            for c in range(D // L):
                a = prow_v[g * L + r, pl.ds(c * L, L)]
                b = rows_v[g * L + r, pl.ds(c * L, L)]
                dlt = a - b
                acc = acc + dlt * dlt
            plsc.store_scatter(tr_v, [lane, jnp.full((L,), r, jnp.int32)], acc)
        tot = jnp.zeros((L,), jnp.float32)
        for r in range(L):
            tot = tot + tr_v[r, :]
        ss_v[pl.ds(g * L, L)] = tot
        return carry

    ngrp = RPW // L
    own_rows.wait()
    ga.wait()
    lax.fori_loop(0, ngrp // 2, ssgroup, 0)
    gb.wait()
    lax.fori_loop(ngrp // 2, ngrp, ssgroup, 0)
    pltpu.sync_copy(ss_v, out_hbm.at[owid])


def _sc_pair_gather(*args):
    return functools.partial(
        pl.kernel,
        out_type=jax.ShapeDtypeStruct((NW, RPW), jnp.float32),
        mesh=plsc.VectorSubcoreMesh(
            core_axis_name="c", subcore_axis_name="s",
            num_cores=NC, num_subcores=NS),
        compiler_params=pltpu.CompilerParams(needs_layout_passes=False),
        scratch_types=[
            pltpu.VMEM((IPW,), jnp.int32),
            pltpu.VMEM((IPW,), jnp.int32),
            pltpu.VMEM((B,), jnp.int32),
            pltpu.VMEM((CPAD,), jnp.int32),
            pltpu.VMEM((CPAD,), jnp.int32),
            pltpu.VMEM((CPAD,), jnp.int32),
            pltpu.VMEM((NS, CPAD), jnp.int32),
            pltpu.VMEM((2 * RPW,), jnp.int32),
            pltpu.VMEM((RPW,), jnp.int32),
            pltpu.VMEM((RPW,), jnp.int32),
            pltpu.VMEM((RPW,), jnp.int32),
            pltpu.VMEM((RPW,), jnp.int32),
            pltpu.VMEM((RPW,), jnp.int32),
            pltpu.VMEM((RPW, D), jnp.float32),
            pltpu.VMEM((RPW, D), jnp.float32),
            pltpu.VMEM((L, L), jnp.float32),
            pltpu.VMEM((RPW,), jnp.float32),
            pltpu.VMEM_SHARED((NS, CPAD), jnp.int32),
            pltpu.VMEM_SHARED((B,), jnp.int32),
            pltpu.SemaphoreType.DMA,
            pltpu.SemaphoreType.DMA,
            pltpu.SemaphoreType.DMA,
        ],
    )(_sc_body)(*args)


def _tc_body(post_t_ref, lab_ref, ss_ref, out_ref):
    ss = ss_ref[...]
    dist = jnp.sum(jnp.sqrt(jnp.where(ss == 0.0, 1e-5, ss)))
    x = post_t_ref[...]
    m = jnp.max(x, axis=0, keepdims=True)
    s = jnp.sum(jnp.exp(x - m), axis=0, keepdims=True)
    lse_sum = jnp.sum(jnp.log(s) + m)
    cls = lax.broadcasted_iota(jnp.int32, (C, 1), 0)
    onehot = lab_ref[...] == cls
    xl_sum = jnp.sum(jnp.where(onehot, x, 0.0))
    out_ref[...] = jnp.reshape((lse_sum - xl_sum) / B + TEMP * dist, (1, 1))


def kernel(pre_projection_activations, post_projection_activations, labels):
    pre = pre_projection_activations
    post = post_projection_activations
    lab32 = labels.astype(jnp.int32)
    hl = jnp.asarray(_HL_W)
    ss = _sc_pair_gather(lab32, hl, pre)
    out = pl.pallas_call(
        _tc_body,
        out_shape=jax.ShapeDtypeStruct((1, 1), jnp.float32),
    )(post.T, lab32.reshape(1, B), ss)
    return out[0, 0]


# speedup vs baseline: 1.0321x; 1.0051x over previous
import functools

import jax
import jax.numpy as jnp
import numpy as np
from jax import lax
from jax.experimental import pallas as pl
from jax.experimental.pallas import tpu as pltpu
from jax.experimental.pallas import tpu_sc as plsc

B = 4096
C = 50
CPAD = 64
D = 256
TEMP = 0.01
NC = 2
NS = 16
L = 16
NW = NC * NS
RPW = B // NW
IPW = B // NS


def _rotl(x, r):
    return ((x << np.uint32(r)) | (x >> np.uint32(32 - r))).astype(np.uint32)


def _tf2x32(k0, k1, x0, x1):
    x0 = x0.astype(np.uint32).copy()
    x1 = x1.astype(np.uint32).copy()
    ks = [k0, k1, np.bitwise_xor(np.bitwise_xor(k0, k1),
                                 np.uint32(0x1BD11BDA)).astype(np.uint32)]
    rotations = [(13, 15, 26, 6), (17, 29, 16, 24)]
    x0 = (x0 + ks[0]).astype(np.uint32)
    x1 = (x1 + ks[1]).astype(np.uint32)
    for i in range(5):
        for r in rotations[i % 2]:
            x0 = (x0 + x1).astype(np.uint32)
            x1 = _rotl(x1, r) ^ x0
        x0 = (x0 + ks[(i + 1) % 3]).astype(np.uint32)
        x1 = (x1 + ks[(i + 2) % 3] + np.uint32(i + 1)).astype(np.uint32)
    return x0, x1


def _pair_words():
    z = np.zeros(B, np.uint32)
    b1, b2 = _tf2x32(np.uint32(0), np.uint32(1), z, np.arange(B, dtype=np.uint32))
    c1a, c2a = _tf2x32(b1, b2, z, z)
    c1b, c2b = _tf2x32(b1, b2, z, np.ones(B, np.uint32))
    h1, h2 = _tf2x32(c1a, c2a, z, z)
    l1, l2 = _tf2x32(c1b, c2b, z, z)
    return (h1 ^ h2).view(np.int32), (l1 ^ l2).view(np.int32)


_HI_W, _LO_W = _pair_words()
_HL_W = np.concatenate(
    [_HI_W.reshape(NW, B // NW), _LO_W.reshape(NW, B // NW)], axis=1)


def _sc_body(labels_hbm, hl_hbm, pre_hbm, out_hbm,
             labels_v, rank_v, m_v, hist_v, offs_v, pret_v,
             allh_v, hlw_v, jidx_v, pos_v, cnt_v, qa_v, qb_v, ida_v, idb_v,
             rows_v, prow_v, tr_v, ss_v, hs_s, ms_s, sem, sem2, sem3):
    cid = lax.axis_index("c")
    sid = lax.axis_index("s")
    ibase = sid * IPW
    obase = cid * RPW
    rbase = ibase + obase
    owid = sid * NC + cid

    pltpu.sync_copy(labels_hbm.at[pl.ds(ibase, IPW)], labels_v)
    pltpu.sync_copy(hl_hbm.at[owid], hlw_v)
    own_rows = pltpu.async_copy(pre_hbm.at[pl.ds(rbase, RPW)], prow_v, sem2)

    lane = lax.iota(jnp.int32, L)
    zeros = jnp.zeros((L,), jnp.int32)
    for c in range(CPAD // L):
        hist_v[pl.ds(c * L, L)] = zeros

    for k in range(IPW // L):
        lbl = labels_v[pl.ds(k * L, L)]
        occ, last = plsc.scan_count(lbl)
        h = plsc.load_gather(hist_v, [lbl])
        rank_v[pl.ds(k * L, L)] = h + occ - 1
        plsc.store_scatter(hist_v, [lbl], h + occ, mask=last)

    pltpu.sync_copy(hist_v, hs_s.at[sid])
    plsc.subcore_barrier()
    pltpu.sync_copy(hs_s, allh_v)
    carry = jnp.int32(0)
    for c in range(CPAD // L):
        tot = zeros
        pre = zeros
        for t in range(NS):
            row = allh_v[t, pl.ds(c * L, L)]
            tot = tot + row
            pre = pre + jnp.where(t < sid, row, zeros)
        hist_v[pl.ds(c * L, L)] = tot
        pret_v[pl.ds(c * L, L)] = pre
        cum = plsc.cumsum(tot)
        offs_v[pl.ds(c * L, L)] = cum - tot + carry
        carry = carry + jnp.sum(tot)

    for k in range(IPW // L):
        lbl = labels_v[pl.ds(k * L, L)]
        g = rank_v[pl.ds(k * L, L)] + plsc.load_gather(pret_v, [lbl])
        rank_v[pl.ds(k * L, L)] = g
        q = plsc.load_gather(offs_v, [lbl]) + g
        half, off = divmod(k * L, RPW)
        qref, idref = (qa_v, ida_v) if half == 0 else (qb_v, idb_v)
        qref[pl.ds(off, L)] = q
        idref[pl.ds(off, L)] = ibase + k * L + lane
    pltpu.sync_copy(ida_v, ms_s.at[qa_v])
    pltpu.sync_copy(idb_v, ms_s.at[qb_v])
    plsc.subcore_barrier()

    for k in range(RPW // L):
        lbl = labels_v[pl.ds(obase + k * L, L)]
        rnk = rank_v[pl.ds(obase + k * L, L)]
        hi = hlw_v[pl.ds(k * L, L)]
        lo = hlw_v[pl.ds(RPW + k * L, L)]
        cnt = plsc.load_gather(hist_v, [lbl]) - 1
        span = jnp.maximum(cnt, 1)
        m1 = lax.rem(jnp.full((L,), 1 << 16, jnp.int32), span)
        mult = lax.rem(m1 * m1, span)

        def u32mod(w, span=span, m1=m1):
            wh = lax.shift_right_logical(w, 16)
            wl = jnp.bitwise_and(w, 0xFFFF)
            return lax.rem(lax.rem(wh, span) * m1 + lax.rem(wl, span), span)

        r = lax.rem(u32mod(hi) * mult + u32mod(lo), span)
        s = r + (r >= rnk).astype(jnp.int32)
        off = plsc.load_gather(offs_v, [lbl])
        pos_v[pl.ds(k * L, L)] = jnp.minimum(off + s, B - 1)
        cnt_v[pl.ds(k * L, L)] = cnt

    pltpu.sync_copy(ms_s.at[pos_v], m_v)
    for k in range(RPW // L):
        cnt = cnt_v[pl.ds(k * L, L)]
        j = m_v[pl.ds(k * L, L)]
        gi = rbase + k * L + lane
        jidx_v[pl.ds(k * L, L)] = jnp.where(cnt == 0, gi, j)

    half = RPW // 2
    ga = pltpu.async_copy(pre_hbm.at[jidx_v.at[pl.ds(0, half)]],
                          rows_v.at[pl.ds(0, half), :], sem)
    gb = pltpu.async_copy(pre_hbm.at[jidx_v.at[pl.ds(half, half)]],
                          rows_v.at[pl.ds(half, half), :], sem3)

    def ssgroup(g, carry):
        for r in range(L):
            acc = jnp.zeros((L,), jnp.float32)
            for c in range(D // L):
                a = prow_v[g * L + r, pl.ds(c * L, L)]
                b = rows_v[g * L + r, pl.ds(c * L, L)]
                dlt = a - b
                acc = acc + dlt * dlt
            plsc.store_scatter(tr_v, [lane, jnp.full((L,), r, jnp.int32)], acc)
        tot = jnp.zeros((L,), jnp.float32)
        for r in range(L):
            tot = tot + tr_v[r, :]
        ss_v[pl.ds(g * L, L)] = tot
        return carry

    ngrp = RPW // L
    own_rows.wait()
    ga.wait()
    lax.fori_loop(0, ngrp // 2, ssgroup, 0)
    gb.wait()
    lax.fori_loop(ngrp // 2, ngrp, ssgroup, 0)
    pltpu.sync_copy(ss_v, out_hbm.at[owid])


def _sc_pair_gather(*args):
    return functools.partial(
        pl.kernel,
        out_type=jax.ShapeDtypeStruct((NW, RPW), jnp.float32),
        mesh=plsc.VectorSubcoreMesh(
            core_axis_name="c", subcore_axis_name="s",
            num_cores=NC, num_subcores=NS),
        compiler_params=pltpu.CompilerParams(needs_layout_passes=False),
        scratch_types=[
            pltpu.VMEM((IPW,), jnp.int32),
            pltpu.VMEM((IPW,), jnp.int32),
            pltpu.VMEM((RPW,), jnp.int32),
            pltpu.VMEM((CPAD,), jnp.int32),
            pltpu.VMEM((CPAD,), jnp.int32),
            pltpu.VMEM((CPAD,), jnp.int32),
            pltpu.VMEM((NS, CPAD), jnp.int32),
            pltpu.VMEM((2 * RPW,), jnp.int32),
            pltpu.VMEM((RPW,), jnp.int32),
            pltpu.VMEM((RPW,), jnp.int32),
            pltpu.VMEM((RPW,), jnp.int32),
            pltpu.VMEM((RPW,), jnp.int32),
            pltpu.VMEM((RPW,), jnp.int32),
            pltpu.VMEM((RPW,), jnp.int32),
            pltpu.VMEM((RPW,), jnp.int32),
            pltpu.VMEM((RPW, D), jnp.float32),
            pltpu.VMEM((RPW, D), jnp.float32),
            pltpu.VMEM((L, L), jnp.float32),
            pltpu.VMEM((RPW,), jnp.float32),
            pltpu.VMEM_SHARED((NS, CPAD), jnp.int32),
            pltpu.VMEM_SHARED((B,), jnp.int32),
            pltpu.SemaphoreType.DMA,
            pltpu.SemaphoreType.DMA,
            pltpu.SemaphoreType.DMA,
        ],
    )(_sc_body)(*args)


def _tc_body(post_t_ref, lab_ref, ss_ref, out_ref):
    ss = ss_ref[...]
    dist = jnp.sum(jnp.sqrt(jnp.where(ss == 0.0, 1e-5, ss)))
    x = post_t_ref[...]
    m = jnp.max(x, axis=0, keepdims=True)
    s = jnp.sum(jnp.exp(x - m), axis=0, keepdims=True)
    lse_sum = jnp.sum(jnp.log(s) + m)
    cls = lax.broadcasted_iota(jnp.int32, (C, 1), 0)
    onehot = lab_ref[...] == cls
    xl_sum = jnp.sum(jnp.where(onehot, x, 0.0))
    out_ref[...] = jnp.reshape((lse_sum - xl_sum) / B + TEMP * dist, (1, 1))


def kernel(pre_projection_activations, post_projection_activations, labels):
    pre = pre_projection_activations
    post = post_projection_activations
    lab32 = labels.astype(jnp.int32)
    hl = jnp.asarray(_HL_W)
    ss = _sc_pair_gather(lab32, hl, pre)
    out = pl.pallas_call(
        _tc_body,
        out_shape=jax.ShapeDtypeStruct((1, 1), jnp.float32),
    )(post.T, lab32.reshape(1, B), ss)
    return out[0, 0]
